# Initial kernel scaffold; baseline (speedup 1.0000x reference)
import jax
import jax.numpy as jnp
from jax.experimental import pallas as pl


def kernel(x, edge_index, edge_attr, node_type_emb, edge_type_emb, W_self, W_msg, W_edge, b):
    raise NotImplementedError("write your pallas kernel here")


# speedup vs baseline: 5.9944x; 5.9944x over previous
import functools
import math

import jax
import jax.numpy as jnp
from jax import lax
from jax.experimental import pallas as pl
from jax.experimental.pallas import tpu as pltpu
from jax.experimental.pallas import tpu_sc as plsc

_ATTR_DIM = 10
_ATTRN = 5
_HIDDEN = 50
_N = 50000
_E = 1600000
_M = 10

_EROWS = _E // 128
_EROWS_PAD = 12544
_TILE_ROWS = _EROWS_PAD // 16
_KB = 4
_NCHUNK = _TILE_ROWS // _KB
_AGG_ROWS = 51200
_ZROWS = _AGG_ROWS // 16

_BN = 1000
_NBLK = _N // _BN
_RB = 112
_EBLK = _EROWS_PAD // _RB


def _pe_consts():
    f = [float(i) * (-math.log(10000.0) / _ATTR_DIM) for i in range(0, _ATTR_DIM, 2)]
    f10 = [f[d // 2] for d in range(_ATTR_DIM)]
    ph = [0.0 if d % 2 == 0 else math.pi / 2.0 for d in range(_ATTR_DIM)]
    return (jnp.asarray(f10, jnp.float32).reshape(1, _ATTR_DIM),
            jnp.asarray(ph, jnp.float32).reshape(1, _ATTR_DIM))


def _node_body(x_ref, emb_ref, eemb_ref, ws_ref, wm_ref, we_ref, b_ref,
               f10_ref, ph_ref,
               hs_ref, g_ref, am_ref, at_ref):
    f10 = f10_ref[...]
    ph = ph_ref[...]
    xb = x_ref[...]
    pieces = []
    for i in range(_ATTRN):
        pieces.append(jnp.sin(xb[:, i:i + 1] * f10 + ph) + emb_ref[i])
    h = jnp.concatenate(pieces, axis=1)
    hs_ref[...] = jnp.dot(h, ws_ref[...], preferred_element_type=jnp.float32)
    hm = jnp.dot(h, wm_ref[...], preferred_element_type=jnp.float32)
    pe0 = jnp.sin(ph)
    pe1 = jnp.sin(f10 + ph)
    ea4 = jnp.concatenate([eemb_ref[0:1] + pe0, eemb_ref[0:1] + pe1,
                           eemb_ref[1:2] + pe0, eemb_ref[1:2] + pe1], axis=0)
    c4 = jnp.dot(ea4, we_ref[...], preferred_element_type=jnp.float32) + b_ref[...]
    for k in range(4):
        g = jnp.maximum(hm + c4[k:k + 1], 0.0)
        g_ref[0, k] = g[:, 0:32]
        g_ref[1, k] = g[:, 18:50]
    for i in range(_M):
        maci = jnp.sin(float(i) * f10 + ph) + emb_ref[2]
        am_ref[i] = jnp.broadcast_to(maci, (xb.shape[0], _ATTR_DIM))
        at_ref[i] = jnp.sin(xb[:, _ATTRN + i:_ATTRN + i + 1] * f10 + ph) + emb_ref[4]


def _edge_body(ei_ref, eidx_ref, dst_ref):
    pid = pl.program_id(0)
    src = ei_ref[0]
    dstv = ei_ref[1]
    a0 = ei_ref[2]
    a1 = ei_ref[3]
    rid = pid * _RB + lax.broadcasted_iota(jnp.int32, (_RB, 128), 0)
    valid = rid < _EROWS
    eidx_ref[...] = jnp.where(valid, (a0 * 2 + a1) * _N + src, 0)
    dst_ref[...] = jnp.where(valid, dstv, _N)


def _final_body(hs_ref, agg_ref, out_ref):
    cat = jnp.concatenate([agg_ref[0, :, 0:25], agg_ref[1, :, 7:32]], axis=1)
    out_ref[...] = jnp.maximum(hs_ref[...] + cat, 0.0)


def _sc_body(g_hbm, eidx_hbm, dst_hbm, zz_hbm, out_hbm,
             eidx_v, dst_v, rows_v, aggm, sem):
    cid = lax.axis_index("c")
    sid = lax.axis_index("s")
    zrow = pl.multiple_of(sid * _ZROWS, 8)
    pltpu.sync_copy(zz_hbm, rows_v.at[0])

    def zloop(i, c):
        r = pl.multiple_of(zrow + i * 128, 8)
        pltpu.sync_copy(rows_v.at[0], aggm.at[pl.ds(r, 128)])
        return c

    lax.fori_loop(0, _ZROWS // 128, zloop, 0)
    plsc.subcore_barrier()
    row0 = sid * _TILE_ROWS
    off = cid * (4 * _N)

    def chunk(ci, carry):
        r = pl.multiple_of(row0 + ci * _KB, 4)
        pltpu.sync_copy(eidx_hbm.at[pl.ds(r, _KB)], eidx_v)
        pltpu.sync_copy(dst_hbm.at[pl.ds(r, _KB)], dst_v)
        for j in range(_KB):
            for i in range(8):
                sl = pl.ds(i * 16, 16)
                eidx_v[j, sl] = eidx_v[j, sl] + off
        descs = [pltpu.async_copy(g_hbm.at[eidx_v.at[j]], rows_v.at[j], sem)
                 for j in range(_KB)]
        for d in descs:
            d.wait()
        for j in range(_KB):
            pltpu.sync_copy(rows_v.at[j], aggm.at[dst_v.at[j]], add=True)
        return carry

    lax.fori_loop(0, _NCHUNK, chunk, 0)
    plsc.subcore_barrier()
    def wloop(i, c):
        r = pl.multiple_of(zrow + i * 128, 8)
        pltpu.sync_copy(aggm.at[pl.ds(r, 128)], rows_v.at[0])
        pltpu.sync_copy(rows_v.at[0], out_hbm.at[cid, pl.ds(r, 128)])
        return c

    lax.fori_loop(0, _ZROWS // 128, wloop, 0)


def _node_call(x, node_type_emb, edge_type_emb, W_self, W_msg, W_edge, b2, f10, ph):
    full = lambda *s: pl.BlockSpec(s, lambda i: tuple(0 for _ in s))
    return pl.pallas_call(
        _node_body,
        grid=(_NBLK,),
        in_specs=[
            pl.BlockSpec((_BN, 15), lambda i: (i, 0)),
            full(_ATTRN, _ATTR_DIM),
            full(2, _ATTR_DIM),
            full(_HIDDEN, _HIDDEN),
            full(_HIDDEN, _HIDDEN),
            full(_ATTR_DIM, _HIDDEN),
            full(1, _HIDDEN),
            full(1, _ATTR_DIM),
            full(1, _ATTR_DIM),
        ],
        out_specs=[
            pl.BlockSpec((_BN, _HIDDEN), lambda i: (i, 0)),
            pl.BlockSpec((2, 4, _BN, 32), lambda i: (0, 0, i, 0)),
            pl.BlockSpec((_M, _BN, _ATTR_DIM), lambda i: (0, i, 0)),
            pl.BlockSpec((_M, _BN, _ATTR_DIM), lambda i: (0, i, 0)),
        ],
        out_shape=[
            jax.ShapeDtypeStruct((_N, _HIDDEN), jnp.float32),
            jax.ShapeDtypeStruct((2, 4, _N, 32), jnp.float32),
            jax.ShapeDtypeStruct((_M, _N, _ATTR_DIM), jnp.float32),
            jax.ShapeDtypeStruct((_M, _N, _ATTR_DIM), jnp.float32),
        ],
    )(x, node_type_emb, edge_type_emb, W_self, W_msg, W_edge, b2, f10, ph)


def _edge_call(ei4):
    return pl.pallas_call(
        _edge_body,
        grid=(_EBLK,),
        in_specs=[pl.BlockSpec((4, _RB, 128), lambda i: (0, i, 0))],
        out_specs=[
            pl.BlockSpec((_RB, 128), lambda i: (i, 0)),
            pl.BlockSpec((_RB, 128), lambda i: (i, 0)),
        ],
        out_shape=[
            jax.ShapeDtypeStruct((_EROWS_PAD, 128), jnp.int32),
            jax.ShapeDtypeStruct((_EROWS_PAD, 128), jnp.int32),
        ],
    )(ei4)


def _final_call(hs, agg):
    return pl.pallas_call(
        _final_body,
        grid=(_NBLK,),
        in_specs=[
            pl.BlockSpec((_BN, _HIDDEN), lambda i: (i, 0)),
            pl.BlockSpec((2, _BN, 32), lambda i: (0, i, 0)),
        ],
        out_specs=pl.BlockSpec((_BN, _HIDDEN), lambda i: (i, 0)),
        out_shape=jax.ShapeDtypeStruct((_N, _HIDDEN), jnp.float32),
    )(hs, agg)


def _make_sc_call():
    return pl.kernel(
        _sc_body,
        out_type=jax.ShapeDtypeStruct((2, _AGG_ROWS, 32), jnp.float32),
        mesh=plsc.VectorSubcoreMesh(core_axis_name="c", subcore_axis_name="s"),
        compiler_params=pltpu.CompilerParams(use_tc_tiling_on_sc=False),
        scratch_types=[
            pltpu.VMEM((_KB, 128), jnp.int32),
            pltpu.VMEM((_KB, 128), jnp.int32),
            pltpu.VMEM((_KB, 128, 32), jnp.float32),
            pltpu.VMEM_SHARED((_AGG_ROWS, 32), jnp.float32),
            pltpu.SemaphoreType.DMA,
        ],
    )


def kernel(x, edge_index, edge_attr, node_type_emb, edge_type_emb,
           W_self, W_msg, W_edge, b):
    f10, ph = _pe_consts()
    b2 = b.reshape(1, _HIDDEN)
    hs, g, am, at = _node_call(
        x, node_type_emb, edge_type_emb, W_self, W_msg, W_edge, b2, f10, ph)
    g = g.reshape(8 * _N, 32)

    ei = edge_index.reshape(2, _EROWS, 128)
    ea = edge_attr.T.reshape(2, _EROWS, 128)
    ei4 = jnp.pad(jnp.concatenate([ei, ea], axis=0),
                  ((0, 0), (0, _EROWS_PAD - _EROWS), (0, 0)))
    eidx2d, dst2d = _edge_call(ei4)

    zz = jnp.zeros((128, 32), jnp.float32)
    agg = _make_sc_call()(g, eidx2d, dst2d, zz)

    out = _final_call(hs, agg)
    return out, (am, at)


# speedup vs baseline: 9.4718x; 1.5801x over previous
import functools
import math

import jax
import jax.numpy as jnp
from jax import lax
from jax.experimental import pallas as pl
from jax.experimental.pallas import tpu as pltpu
from jax.experimental.pallas import tpu_sc as plsc

_ATTR_DIM = 10
_ATTRN = 5
_HIDDEN = 50
_N = 50000
_E = 1600000
_M = 10

_EROWS = _E // 128
_EROWS_PAD = 12544
_TILE_ROWS = _EROWS_PAD // 16
_KB = 4
_NCHUNK = _TILE_ROWS // _KB
_AGG_ROWS = 51200
_ZROWS = _AGG_ROWS // 16

_BN = 1000
_NBLK = _N // _BN
_RB = 112
_EBLK = _EROWS_PAD // _RB


def _pe_consts():
    f = [float(i) * (-math.log(10000.0) / _ATTR_DIM) for i in range(0, _ATTR_DIM, 2)]
    f10 = [f[d // 2] for d in range(_ATTR_DIM)]
    ph = [0.0 if d % 2 == 0 else math.pi / 2.0 for d in range(_ATTR_DIM)]
    return (jnp.asarray(f10, jnp.float32).reshape(1, _ATTR_DIM),
            jnp.asarray(ph, jnp.float32).reshape(1, _ATTR_DIM))


def _pack_consts():
    import numpy as np
    f = [float(i) * (-math.log(10000.0) / _ATTR_DIM) for i in range(0, _ATTR_DIM, 2)]
    f10 = [f[d // 2] for d in range(_ATTR_DIM)]
    ph = [0.0 if d % 2 == 0 else math.pi / 2.0 for d in range(_ATTR_DIM)]
    sel = np.zeros((15, 120), np.float32)
    ph120 = np.zeros((1, 120), np.float32)
    for i in range(15):
        for j in range(8):
            sel[i, 8 * i + j] = f10[2 + j]
            ph120[0, 8 * i + j] = ph[2 + j]
    p50 = np.zeros((120, 50), np.float32)
    base50 = np.zeros((1, 50), np.float32)
    for i in range(5):
        base50[0, 10 * i + 1] = 1.0
        for j in range(8):
            p50[8 * i + j, 10 * i + 2 + j] = 1.0
    pt100 = np.zeros((120, 100), np.float32)
    base100 = np.zeros((1, 100), np.float32)
    for i in range(10):
        base100[0, 10 * i + 1] = 1.0
        for j in range(8):
            pt100[40 + 8 * i + j, 10 * i + 2 + j] = 1.0
    return (jnp.asarray(sel), jnp.asarray(ph120), jnp.asarray(p50),
            jnp.asarray(base50), jnp.asarray(pt100), jnp.asarray(base100))


def _node_body(x_ref, emb_ref, eemb_ref, ws_ref, wm_ref, we_ref, b_ref,
               f10_ref, ph_ref, sel_ref, ph120_ref, p50_ref, base50_ref,
               pt100_ref, base100_ref,
               hs_ref, g_ref, am_ref, at_ref):
    f10 = f10_ref[...]
    ph = ph_ref[...]
    xb = x_ref[...]
    bn = xb.shape[0]
    ang = jnp.dot(xb, sel_ref[...], preferred_element_type=jnp.float32, precision=lax.Precision.HIGHEST) + ph120_ref[...]
    s = jnp.sin(ang)
    embcat = jnp.concatenate([emb_ref[i:i + 1] for i in range(_ATTRN)], axis=1)
    h = (jnp.dot(s, p50_ref[...], preferred_element_type=jnp.float32, precision=lax.Precision.HIGHEST)
         + base50_ref[...] + embcat)
    hs_ref[...] = jnp.dot(h, ws_ref[...], preferred_element_type=jnp.float32, precision=lax.Precision.HIGHEST)
    hm = jnp.dot(h, wm_ref[...], preferred_element_type=jnp.float32, precision=lax.Precision.HIGHEST)
    pe0 = jnp.sin(ph)
    pe1 = jnp.sin(f10 + ph)
    ea4 = jnp.concatenate([eemb_ref[0:1] + pe0, eemb_ref[0:1] + pe1,
                           eemb_ref[1:2] + pe0, eemb_ref[1:2] + pe1], axis=0)
    c4 = jnp.dot(ea4, we_ref[...], preferred_element_type=jnp.float32, precision=lax.Precision.HIGHEST) + b_ref[...]
    for k in range(4):
        g = jnp.maximum(hm + c4[k:k + 1], 0.0)
        g_ref[0, k] = g[:, 0:32]
        g_ref[1, k] = g[:, 18:50]
    emb4t = jnp.concatenate([emb_ref[4:5]] * _M, axis=1)
    at100 = (jnp.dot(s, pt100_ref[...], preferred_element_type=jnp.float32, precision=lax.Precision.HIGHEST)
             + base100_ref[...] + emb4t)
    for i in range(_M):
        maci = jnp.sin(float(i) * f10 + ph) + emb_ref[2]
        am_ref[i] = jnp.broadcast_to(maci, (bn, _ATTR_DIM))
        at_ref[i] = at100[:, 10 * i:10 * i + 10]


def _edge_body(ei_ref, eidx_ref, dst_ref):
    pid = pl.program_id(0)
    src = ei_ref[0]
    dstv = ei_ref[1]
    a0 = ei_ref[2]
    a1 = ei_ref[3]
    rid = pid * _RB + lax.broadcasted_iota(jnp.int32, (_RB, 128), 0)
    valid = rid < _EROWS
    eidx_ref[...] = jnp.where(valid, (a0 * 2 + a1) * _N + src, 0)
    dst_ref[...] = jnp.where(valid, dstv, _N)


def _final_body(hs_ref, agg_ref, out_ref):
    cat = jnp.concatenate([agg_ref[0, :, 0:25], agg_ref[1, :, 7:32]], axis=1)
    out_ref[...] = jnp.maximum(hs_ref[...] + cat, 0.0)


def _sc_body(g_hbm, eidx_hbm, dst_hbm, zz_hbm, out_hbm,
             eidx_v, dst_v, rows_v, aggm, sem):
    cid = lax.axis_index("c")
    sid = lax.axis_index("s")
    zrow = pl.multiple_of(sid * _ZROWS, 8)
    pltpu.sync_copy(zz_hbm, rows_v.at[0])

    def zloop(i, c):
        r = pl.multiple_of(zrow + i * 128, 8)
        pltpu.sync_copy(rows_v.at[0], aggm.at[pl.ds(r, 128)])
        return c

    lax.fori_loop(0, _ZROWS // 128, zloop, 0)
    plsc.subcore_barrier()
    row0 = sid * _TILE_ROWS
    off = cid * (4 * _N)

    def chunk(ci, carry):
        r = pl.multiple_of(row0 + ci * _KB, 4)
        pltpu.sync_copy(eidx_hbm.at[pl.ds(r, _KB)], eidx_v)
        pltpu.sync_copy(dst_hbm.at[pl.ds(r, _KB)], dst_v)
        for j in range(_KB):
            for i in range(8):
                sl = pl.ds(i * 16, 16)
                eidx_v[j, sl] = eidx_v[j, sl] + off
        descs = [pltpu.async_copy(g_hbm.at[eidx_v.at[j]], rows_v.at[j], sem)
                 for j in range(_KB)]
        for d in descs:
            d.wait()
        for j in range(_KB):
            pltpu.sync_copy(rows_v.at[j], aggm.at[dst_v.at[j]], add=True)
        return carry

    lax.fori_loop(0, _NCHUNK, chunk, 0)
    plsc.subcore_barrier()
    def wloop(i, c):
        r = pl.multiple_of(zrow + i * 128, 8)
        pltpu.sync_copy(aggm.at[pl.ds(r, 128)], rows_v.at[0])
        pltpu.sync_copy(rows_v.at[0], out_hbm.at[cid, pl.ds(r, 128)])
        return c

    lax.fori_loop(0, _ZROWS // 128, wloop, 0)


def _node_call(x, node_type_emb, edge_type_emb, W_self, W_msg, W_edge, b2, f10, ph):
    sel, ph120, p50, base50, pt100, base100 = _pack_consts()
    full = lambda *s: pl.BlockSpec(s, lambda i: tuple(0 for _ in s))
    return pl.pallas_call(
        _node_body,
        grid=(_NBLK,),
        in_specs=[
            pl.BlockSpec((_BN, 15), lambda i: (i, 0)),
            full(_ATTRN, _ATTR_DIM),
            full(2, _ATTR_DIM),
            full(_HIDDEN, _HIDDEN),
            full(_HIDDEN, _HIDDEN),
            full(_ATTR_DIM, _HIDDEN),
            full(1, _HIDDEN),
            full(1, _ATTR_DIM),
            full(1, _ATTR_DIM),
            full(15, 120),
            full(1, 120),
            full(120, _HIDDEN),
            full(1, _HIDDEN),
            full(120, 100),
            full(1, 100),
        ],
        out_specs=[
            pl.BlockSpec((_BN, _HIDDEN), lambda i: (i, 0)),
            pl.BlockSpec((2, 4, _BN, 32), lambda i: (0, 0, i, 0)),
            pl.BlockSpec((_M, _BN, _ATTR_DIM), lambda i: (0, i, 0)),
            pl.BlockSpec((_M, _BN, _ATTR_DIM), lambda i: (0, i, 0)),
        ],
        out_shape=[
            jax.ShapeDtypeStruct((_N, _HIDDEN), jnp.float32),
            jax.ShapeDtypeStruct((2, 4, _N, 32), jnp.float32),
            jax.ShapeDtypeStruct((_M, _N, _ATTR_DIM), jnp.float32),
            jax.ShapeDtypeStruct((_M, _N, _ATTR_DIM), jnp.float32),
        ],
    )(x, node_type_emb, edge_type_emb, W_self, W_msg, W_edge, b2, f10, ph,
      sel, ph120, p50, base50, pt100, base100)


def _edge_call(ei4):
    return pl.pallas_call(
        _edge_body,
        grid=(_EBLK,),
        in_specs=[pl.BlockSpec((4, _RB, 128), lambda i: (0, i, 0))],
        out_specs=[
            pl.BlockSpec((_RB, 128), lambda i: (i, 0)),
            pl.BlockSpec((_RB, 128), lambda i: (i, 0)),
        ],
        out_shape=[
            jax.ShapeDtypeStruct((_EROWS_PAD, 128), jnp.int32),
            jax.ShapeDtypeStruct((_EROWS_PAD, 128), jnp.int32),
        ],
    )(ei4)


def _final_call(hs, agg):
    return pl.pallas_call(
        _final_body,
        grid=(_NBLK,),
        in_specs=[
            pl.BlockSpec((_BN, _HIDDEN), lambda i: (i, 0)),
            pl.BlockSpec((2, _BN, 32), lambda i: (0, i, 0)),
        ],
        out_specs=pl.BlockSpec((_BN, _HIDDEN), lambda i: (i, 0)),
        out_shape=jax.ShapeDtypeStruct((_N, _HIDDEN), jnp.float32),
    )(hs, agg)


def _make_sc_call():
    return pl.kernel(
        _sc_body,
        out_type=jax.ShapeDtypeStruct((2, _AGG_ROWS, 32), jnp.float32),
        mesh=plsc.VectorSubcoreMesh(core_axis_name="c", subcore_axis_name="s"),
        compiler_params=pltpu.CompilerParams(use_tc_tiling_on_sc=False),
        scratch_types=[
            pltpu.VMEM((_KB, 128), jnp.int32),
            pltpu.VMEM((_KB, 128), jnp.int32),
            pltpu.VMEM((_KB, 128, 32), jnp.float32),
            pltpu.VMEM_SHARED((_AGG_ROWS, 32), jnp.float32),
            pltpu.SemaphoreType.DMA,
        ],
    )


def kernel(x, edge_index, edge_attr, node_type_emb, edge_type_emb,
           W_self, W_msg, W_edge, b):
    f10, ph = _pe_consts()
    b2 = b.reshape(1, _HIDDEN)
    hs, g, am, at = _node_call(
        x, node_type_emb, edge_type_emb, W_self, W_msg, W_edge, b2, f10, ph)
    g = g.reshape(8 * _N, 32)

    ei = edge_index.reshape(2, _EROWS, 128)
    ea = edge_attr.T.reshape(2, _EROWS, 128)
    ei4 = jnp.pad(jnp.concatenate([ei, ea], axis=0),
                  ((0, 0), (0, _EROWS_PAD - _EROWS), (0, 0)))
    eidx2d, dst2d = _edge_call(ei4)

    zz = jnp.zeros((128, 32), jnp.float32)
    agg = _make_sc_call()(g, eidx2d, dst2d, zz)

    out = _final_call(hs, agg)
    return out, (am, at)


# speedup vs baseline: 11.0716x; 1.1689x over previous
import functools
import math

import jax
import jax.numpy as jnp
from jax import lax
from jax.experimental import pallas as pl
from jax.experimental.pallas import tpu as pltpu
from jax.experimental.pallas import tpu_sc as plsc

_ATTR_DIM = 10
_ATTRN = 5
_HIDDEN = 50
_N = 50000
_E = 1600000
_M = 10

_EROWS = _E // 128
_EROWS_PAD = 12544
_TILE_ROWS = _EROWS_PAD // 16
_KB = 4
_NCHUNK = _TILE_ROWS // _KB
_AGG_ROWS = 51200
_ZROWS = _AGG_ROWS // 16

_BN = 2000
_NBLK = _N // _BN
_RB = 112
_EBLK = _EROWS_PAD // _RB


def _pe_consts():
    f = [float(i) * (-math.log(10000.0) / _ATTR_DIM) for i in range(0, _ATTR_DIM, 2)]
    f10 = [f[d // 2] for d in range(_ATTR_DIM)]
    ph = [0.0 if d % 2 == 0 else math.pi / 2.0 for d in range(_ATTR_DIM)]
    return (jnp.asarray(f10, jnp.float32).reshape(1, _ATTR_DIM),
            jnp.asarray(ph, jnp.float32).reshape(1, _ATTR_DIM))


def _pack_consts():
    import numpy as np
    f = [float(i) * (-math.log(10000.0) / _ATTR_DIM) for i in range(0, _ATTR_DIM, 2)]
    f10 = [f[d // 2] for d in range(_ATTR_DIM)]
    ph = [0.0 if d % 2 == 0 else math.pi / 2.0 for d in range(_ATTR_DIM)]
    sel = np.zeros((15, 120), np.float32)
    ph120 = np.zeros((1, 120), np.float32)
    for i in range(15):
        for j in range(8):
            sel[i, 8 * i + j] = f10[2 + j]
            ph120[0, 8 * i + j] = ph[2 + j]
    p50 = np.zeros((120, 50), np.float32)
    base50 = np.zeros((1, 50), np.float32)
    for i in range(5):
        base50[0, 10 * i + 1] = 1.0
        for j in range(8):
            p50[8 * i + j, 10 * i + 2 + j] = 1.0
    pt100 = np.zeros((120, 100), np.float32)
    base100 = np.zeros((1, 100), np.float32)
    for i in range(10):
        base100[0, 10 * i + 1] = 1.0
        for j in range(8):
            pt100[40 + 8 * i + j, 10 * i + 2 + j] = 1.0
    return (jnp.asarray(sel), jnp.asarray(ph120), jnp.asarray(p50),
            jnp.asarray(base50), jnp.asarray(pt100), jnp.asarray(base100))


def _node_body(x_ref, emb_ref, eemb_ref, ws_ref, wm_ref, we_ref, b_ref,
               f10_ref, ph_ref, sel_ref, ph120_ref, p50_ref, base50_ref,
               pt100_ref, base100_ref,
               hs_ref, g_ref, mac_ref, at2_ref):
    f10 = f10_ref[...]
    ph = ph_ref[...]
    xb = x_ref[...]
    bn = xb.shape[0]
    ang = jnp.dot(xb, sel_ref[...], preferred_element_type=jnp.float32, precision=lax.Precision.HIGHEST) + ph120_ref[...]
    s = jnp.sin(ang)
    embcat = jnp.concatenate([emb_ref[i:i + 1] for i in range(_ATTRN)], axis=1)
    h = (jnp.dot(s, p50_ref[...], preferred_element_type=jnp.float32, precision=lax.Precision.HIGHEST)
         + base50_ref[...] + embcat)
    hs_ref[...] = jnp.dot(h, ws_ref[...], preferred_element_type=jnp.float32, precision=lax.Precision.HIGHEST)
    hm = jnp.dot(h, wm_ref[...], preferred_element_type=jnp.float32, precision=lax.Precision.HIGHEST)
    pe0 = jnp.sin(ph)
    pe1 = jnp.sin(f10 + ph)
    ea4 = jnp.concatenate([eemb_ref[0:1] + pe0, eemb_ref[0:1] + pe1,
                           eemb_ref[1:2] + pe0, eemb_ref[1:2] + pe1], axis=0)
    c4 = jnp.dot(ea4, we_ref[...], preferred_element_type=jnp.float32, precision=lax.Precision.HIGHEST) + b_ref[...]
    for k in range(4):
        g = jnp.maximum(hm + c4[k:k + 1], 0.0)
        g_ref[0, k] = g[:, 0:32]
        g_ref[1, k] = g[:, 18:50]
    emb4t = jnp.concatenate([emb_ref[4:5]] * _M, axis=1)
    at2_ref[...] = (jnp.dot(s, pt100_ref[...], preferred_element_type=jnp.float32, precision=lax.Precision.HIGHEST)
                    + base100_ref[...] + emb4t)
    mac_ref[...] = jnp.concatenate(
        [jnp.sin(float(i) * f10 + ph) + emb_ref[2] for i in range(_M)], axis=0)


def _edge_body(ei_ref, eidx_ref, dst_ref):
    pid = pl.program_id(0)
    src = ei_ref[0]
    dstv = ei_ref[1]
    a0 = ei_ref[2]
    a1 = ei_ref[3]
    rid = pid * _RB + lax.broadcasted_iota(jnp.int32, (_RB, 128), 0)
    valid = rid < _EROWS
    eidx_ref[...] = jnp.where(valid, (a0 * 2 + a1) * _N + src, 0)
    dst_ref[...] = jnp.where(valid, dstv, _N)


def _final_body(hs_ref, agg_ref, out_ref):
    cat = jnp.concatenate([agg_ref[0, :, 0:25], agg_ref[1, :, 7:32]], axis=1)
    out_ref[...] = jnp.maximum(hs_ref[...] + cat, 0.0)


def _sc_body(g_hbm, eidx_hbm, dst_hbm, zz_hbm, out_hbm,
             eidx_v, dst_v, rows_v, aggm, sem):
    cid = lax.axis_index("c")
    sid = lax.axis_index("s")
    zrow = pl.multiple_of(sid * _ZROWS, 8)
    pltpu.sync_copy(zz_hbm, rows_v.at[0])

    def zloop(i, c):
        r = pl.multiple_of(zrow + i * 128, 8)
        pltpu.sync_copy(rows_v.at[0], aggm.at[pl.ds(r, 128)])
        return c

    lax.fori_loop(0, _ZROWS // 128, zloop, 0)
    plsc.subcore_barrier()
    row0 = sid * _TILE_ROWS
    off = cid * (4 * _N)

    def chunk(ci, carry):
        r = pl.multiple_of(row0 + ci * _KB, 4)
        pltpu.sync_copy(eidx_hbm.at[pl.ds(r, _KB)], eidx_v)
        pltpu.sync_copy(dst_hbm.at[pl.ds(r, _KB)], dst_v)
        for j in range(_KB):
            for i in range(8):
                sl = pl.ds(i * 16, 16)
                eidx_v[j, sl] = eidx_v[j, sl] + off
        descs = [pltpu.async_copy(g_hbm.at[eidx_v.at[j]], rows_v.at[j], sem)
                 for j in range(_KB)]
        for d in descs:
            d.wait()
        for j in range(_KB):
            pltpu.sync_copy(rows_v.at[j], aggm.at[dst_v.at[j]], add=True)
        return carry

    lax.fori_loop(0, _NCHUNK, chunk, 0)
    plsc.subcore_barrier()
    def wloop(i, c):
        r = pl.multiple_of(zrow + i * 128, 8)
        pltpu.sync_copy(aggm.at[pl.ds(r, 128)], rows_v.at[0])
        pltpu.sync_copy(rows_v.at[0], out_hbm.at[cid, pl.ds(r, 128)])
        return c

    lax.fori_loop(0, _ZROWS // 128, wloop, 0)


def _node_call(x, node_type_emb, edge_type_emb, W_self, W_msg, W_edge, b2, f10, ph):
    sel, ph120, p50, base50, pt100, base100 = _pack_consts()
    full = lambda *s: pl.BlockSpec(s, lambda i: tuple(0 for _ in s))
    return pl.pallas_call(
        _node_body,
        grid=(_NBLK,),
        in_specs=[
            pl.BlockSpec((_BN, 15), lambda i: (i, 0)),
            full(_ATTRN, _ATTR_DIM),
            full(2, _ATTR_DIM),
            full(_HIDDEN, _HIDDEN),
            full(_HIDDEN, _HIDDEN),
            full(_ATTR_DIM, _HIDDEN),
            full(1, _HIDDEN),
            full(1, _ATTR_DIM),
            full(1, _ATTR_DIM),
            full(15, 120),
            full(1, 120),
            full(120, _HIDDEN),
            full(1, _HIDDEN),
            full(120, 100),
            full(1, 100),
        ],
        out_specs=[
            pl.BlockSpec((_BN, _HIDDEN), lambda i: (i, 0)),
            pl.BlockSpec((2, 4, _BN, 32), lambda i: (0, 0, i, 0)),
            pl.BlockSpec((_M, _ATTR_DIM), lambda i: (0, 0)),
            pl.BlockSpec((_BN, 100), lambda i: (i, 0)),
        ],
        out_shape=[
            jax.ShapeDtypeStruct((_N, _HIDDEN), jnp.float32),
            jax.ShapeDtypeStruct((2, 4, _N, 32), jnp.float32),
            jax.ShapeDtypeStruct((_M, _ATTR_DIM), jnp.float32),
            jax.ShapeDtypeStruct((_N, 100), jnp.float32),
        ],
    )(x, node_type_emb, edge_type_emb, W_self, W_msg, W_edge, b2, f10, ph,
      sel, ph120, p50, base50, pt100, base100)


def _edge_call(ei4):
    return pl.pallas_call(
        _edge_body,
        grid=(_EBLK,),
        in_specs=[pl.BlockSpec((4, _RB, 128), lambda i: (0, i, 0))],
        out_specs=[
            pl.BlockSpec((_RB, 128), lambda i: (i, 0)),
            pl.BlockSpec((_RB, 128), lambda i: (i, 0)),
        ],
        out_shape=[
            jax.ShapeDtypeStruct((_EROWS_PAD, 128), jnp.int32),
            jax.ShapeDtypeStruct((_EROWS_PAD, 128), jnp.int32),
        ],
    )(ei4)


def _final_call(hs, agg):
    return pl.pallas_call(
        _final_body,
        grid=(_NBLK,),
        in_specs=[
            pl.BlockSpec((_BN, _HIDDEN), lambda i: (i, 0)),
            pl.BlockSpec((2, _BN, 32), lambda i: (0, i, 0)),
        ],
        out_specs=pl.BlockSpec((_BN, _HIDDEN), lambda i: (i, 0)),
        out_shape=jax.ShapeDtypeStruct((_N, _HIDDEN), jnp.float32),
    )(hs, agg)


def _make_sc_call():
    return pl.kernel(
        _sc_body,
        out_type=jax.ShapeDtypeStruct((2, _AGG_ROWS, 32), jnp.float32),
        mesh=plsc.VectorSubcoreMesh(core_axis_name="c", subcore_axis_name="s"),
        compiler_params=pltpu.CompilerParams(use_tc_tiling_on_sc=False),
        scratch_types=[
            pltpu.VMEM((_KB, 128), jnp.int32),
            pltpu.VMEM((_KB, 128), jnp.int32),
            pltpu.VMEM((_KB, 128, 32), jnp.float32),
            pltpu.VMEM_SHARED((_AGG_ROWS, 32), jnp.float32),
            pltpu.SemaphoreType.DMA,
        ],
    )


def kernel(x, edge_index, edge_attr, node_type_emb, edge_type_emb,
           W_self, W_msg, W_edge, b):
    f10, ph = _pe_consts()
    b2 = b.reshape(1, _HIDDEN)
    hs, g, mac10, at2 = _node_call(
        x, node_type_emb, edge_type_emb, W_self, W_msg, W_edge, b2, f10, ph)
    g = g.reshape(8 * _N, 32)
    am = jnp.broadcast_to(mac10[:, None, :], (_M, _N, _ATTR_DIM))
    at = at2.reshape(_N, _M, _ATTR_DIM).transpose(1, 0, 2)

    ei = edge_index.reshape(2, _EROWS, 128)
    ea = edge_attr.T.reshape(2, _EROWS, 128)
    ei4 = jnp.pad(jnp.concatenate([ei, ea], axis=0),
                  ((0, 0), (0, _EROWS_PAD - _EROWS), (0, 0)))
    eidx2d, dst2d = _edge_call(ei4)

    zz = jnp.zeros((128, 32), jnp.float32)
    agg = _make_sc_call()(g, eidx2d, dst2d, zz)

    out = _final_call(hs, agg)
    return out, (am, at)


# speedup vs baseline: 11.6067x; 1.0483x over previous
import functools
import math

import jax
import jax.numpy as jnp
from jax import lax
from jax.experimental import pallas as pl
from jax.experimental.pallas import tpu as pltpu
from jax.experimental.pallas import tpu_sc as plsc

_ATTR_DIM = 10
_ATTRN = 5
_HIDDEN = 50
_N = 50000
_E = 1600000
_M = 10

_EROWS = _E // 128
_EROWS_PAD = 12544
_TILE_ROWS = _EROWS_PAD // 16
_KB = 2
_NCHUNK = _TILE_ROWS // _KB
_AGG_ROWS = 51200
_ZROWS = _AGG_ROWS // 16

_BN = 2000
_NBLK = _N // _BN
_RB = 112
_EBLK = _EROWS_PAD // _RB


def _pe_consts():
    f = [float(i) * (-math.log(10000.0) / _ATTR_DIM) for i in range(0, _ATTR_DIM, 2)]
    f10 = [f[d // 2] for d in range(_ATTR_DIM)]
    ph = [0.0 if d % 2 == 0 else math.pi / 2.0 for d in range(_ATTR_DIM)]
    return (jnp.asarray(f10, jnp.float32).reshape(1, _ATTR_DIM),
            jnp.asarray(ph, jnp.float32).reshape(1, _ATTR_DIM))


def _pack_consts():
    import numpy as np
    f = [float(i) * (-math.log(10000.0) / _ATTR_DIM) for i in range(0, _ATTR_DIM, 2)]
    f10 = [f[d // 2] for d in range(_ATTR_DIM)]
    ph = [0.0 if d % 2 == 0 else math.pi / 2.0 for d in range(_ATTR_DIM)]
    sel = np.zeros((15, 120), np.float32)
    ph120 = np.zeros((1, 120), np.float32)
    for i in range(15):
        for j in range(8):
            sel[i, 8 * i + j] = f10[2 + j]
            ph120[0, 8 * i + j] = ph[2 + j]
    p50 = np.zeros((120, 50), np.float32)
    base50 = np.zeros((1, 50), np.float32)
    for i in range(5):
        base50[0, 10 * i + 1] = 1.0
        for j in range(8):
            p50[8 * i + j, 10 * i + 2 + j] = 1.0
    pt100 = np.zeros((120, 100), np.float32)
    base100 = np.zeros((1, 100), np.float32)
    for i in range(10):
        base100[0, 10 * i + 1] = 1.0
        for j in range(8):
            pt100[40 + 8 * i + j, 10 * i + 2 + j] = 1.0
    return (jnp.asarray(sel), jnp.asarray(ph120), jnp.asarray(p50),
            jnp.asarray(base50), jnp.asarray(pt100), jnp.asarray(base100))


def _node_body(x_ref, emb_ref, eemb_ref, ws_ref, wm_ref, we_ref, b_ref,
               f10_ref, ph_ref, sel_ref, ph120_ref, p50_ref, base50_ref,
               pt100_ref, base100_ref,
               hs_ref, g_ref, mac_ref, at2_ref):
    f10 = f10_ref[...]
    ph = ph_ref[...]
    xb = x_ref[...]
    bn = xb.shape[0]
    ang = jnp.dot(xb, sel_ref[...], preferred_element_type=jnp.float32, precision=lax.Precision.HIGHEST) + ph120_ref[...]
    s = jnp.sin(ang)
    embcat = jnp.concatenate([emb_ref[i:i + 1] for i in range(_ATTRN)], axis=1)
    h = (jnp.dot(s, p50_ref[...], preferred_element_type=jnp.float32, precision=lax.Precision.HIGHEST)
         + base50_ref[...] + embcat)
    hs_ref[...] = jnp.dot(h, ws_ref[...], preferred_element_type=jnp.float32, precision=lax.Precision.HIGHEST)
    hm = jnp.dot(h, wm_ref[...], preferred_element_type=jnp.float32, precision=lax.Precision.HIGHEST)
    pe0 = jnp.sin(ph)
    pe1 = jnp.sin(f10 + ph)
    ea4 = jnp.concatenate([eemb_ref[0:1] + pe0, eemb_ref[0:1] + pe1,
                           eemb_ref[1:2] + pe0, eemb_ref[1:2] + pe1], axis=0)
    c4 = jnp.dot(ea4, we_ref[...], preferred_element_type=jnp.float32, precision=lax.Precision.HIGHEST) + b_ref[...]
    for k in range(4):
        g = jnp.maximum(hm + c4[k:k + 1], 0.0)
        g_ref[0, k] = g[:, 0:32]
        g_ref[1, k] = g[:, 18:50]
    emb4t = jnp.concatenate([emb_ref[4:5]] * _M, axis=1)
    at2_ref[...] = (jnp.dot(s, pt100_ref[...], preferred_element_type=jnp.float32, precision=lax.Precision.HIGHEST)
                    + base100_ref[...] + emb4t)
    mac_ref[...] = jnp.concatenate(
        [jnp.sin(float(i) * f10 + ph) + emb_ref[2] for i in range(_M)], axis=0)


def _edge_body(ei_ref, eidx_ref, dst_ref):
    pid = pl.program_id(0)
    src = ei_ref[0]
    dstv = ei_ref[1]
    a0 = ei_ref[2]
    a1 = ei_ref[3]
    rid = pid * _RB + lax.broadcasted_iota(jnp.int32, (_RB, 128), 0)
    valid = rid < _EROWS
    eidx_ref[...] = jnp.where(valid, (a0 * 2 + a1) * _N + src, 0)
    dst_ref[...] = jnp.where(valid, dstv, _N)


def _final_body(hs_ref, agg_ref, out_ref):
    cat = jnp.concatenate([agg_ref[0, :, 0:25], agg_ref[1, :, 7:32]], axis=1)
    out_ref[...] = jnp.maximum(hs_ref[...] + cat, 0.0)


def _sc_body(g_hbm, eidx_hbm, dst_hbm, zz_hbm, out_hbm,
             eidx_v, dst_v, rows_v, zz_v, aggm, sem):
    cid = lax.axis_index("c")
    sid = lax.axis_index("s")
    zrow = pl.multiple_of(sid * _ZROWS, 8)
    pltpu.sync_copy(zz_hbm, zz_v)

    def zloop(i, c):
        r = pl.multiple_of(zrow + i * 128, 8)
        pltpu.sync_copy(zz_v, aggm.at[pl.ds(r, 128)])
        return c

    lax.fori_loop(0, _ZROWS // 128, zloop, 0)
    plsc.subcore_barrier()
    row0 = sid * _TILE_ROWS
    off = cid * (4 * _N)

    def fire(buf, ci):
        r = pl.multiple_of(row0 + ci * _KB, 2)
        pltpu.sync_copy(eidx_hbm.at[pl.ds(r, _KB)], eidx_v.at[buf])
        for j in range(_KB):
            for i in range(8):
                sl = pl.ds(i * 16, 16)
                eidx_v[buf, j, sl] = eidx_v[buf, j, sl] + off
        for j in range(_KB):
            pltpu.async_copy(g_hbm.at[eidx_v.at[buf].at[j]],
                             rows_v.at[buf].at[j], sem)

    def drain():
        for j in range(_KB):
            pltpu.make_async_copy(zz_hbm, rows_v.at[0].at[j], sem).wait()

    def scatter(buf, ci):
        r = pl.multiple_of(row0 + ci * _KB, 2)
        pltpu.sync_copy(dst_hbm.at[pl.ds(r, _KB)], dst_v)
        for j in range(_KB):
            pltpu.sync_copy(rows_v.at[buf].at[j], aggm.at[dst_v.at[j]],
                            add=True)

    fire(0, 0)

    def pair(i, carry):
        c0 = 2 * i
        fire(1, c0 + 1)
        drain()
        scatter(0, c0)
        fire(0, c0 + 2)
        drain()
        scatter(1, c0 + 1)
        return carry

    lax.fori_loop(0, _NCHUNK // 2, pair, 0)
    drain()
    plsc.subcore_barrier()
    def wloop(i, c):
        r = pl.multiple_of(zrow + i * 128, 8)
        pltpu.sync_copy(aggm.at[pl.ds(r, 128)], zz_v)
        pltpu.sync_copy(zz_v, out_hbm.at[cid, pl.ds(r, 128)])
        return c

    lax.fori_loop(0, _ZROWS // 128, wloop, 0)


def _node_call(x, node_type_emb, edge_type_emb, W_self, W_msg, W_edge, b2, f10, ph):
    sel, ph120, p50, base50, pt100, base100 = _pack_consts()
    full = lambda *s: pl.BlockSpec(s, lambda i: tuple(0 for _ in s))
    return pl.pallas_call(
        _node_body,
        grid=(_NBLK,),
        in_specs=[
            pl.BlockSpec((_BN, 15), lambda i: (i, 0)),
            full(_ATTRN, _ATTR_DIM),
            full(2, _ATTR_DIM),
            full(_HIDDEN, _HIDDEN),
            full(_HIDDEN, _HIDDEN),
            full(_ATTR_DIM, _HIDDEN),
            full(1, _HIDDEN),
            full(1, _ATTR_DIM),
            full(1, _ATTR_DIM),
            full(15, 120),
            full(1, 120),
            full(120, _HIDDEN),
            full(1, _HIDDEN),
            full(120, 100),
            full(1, 100),
        ],
        out_specs=[
            pl.BlockSpec((_BN, _HIDDEN), lambda i: (i, 0)),
            pl.BlockSpec((2, 4, _BN, 32), lambda i: (0, 0, i, 0)),
            pl.BlockSpec((_M, _ATTR_DIM), lambda i: (0, 0)),
            pl.BlockSpec((_BN, 100), lambda i: (i, 0)),
        ],
        out_shape=[
            jax.ShapeDtypeStruct((_N, _HIDDEN), jnp.float32),
            jax.ShapeDtypeStruct((2, 4, _N, 32), jnp.float32),
            jax.ShapeDtypeStruct((_M, _ATTR_DIM), jnp.float32),
            jax.ShapeDtypeStruct((_N, 100), jnp.float32),
        ],
    )(x, node_type_emb, edge_type_emb, W_self, W_msg, W_edge, b2, f10, ph,
      sel, ph120, p50, base50, pt100, base100)


def _edge_call(ei4):
    return pl.pallas_call(
        _edge_body,
        grid=(_EBLK,),
        in_specs=[pl.BlockSpec((4, _RB, 128), lambda i: (0, i, 0))],
        out_specs=[
            pl.BlockSpec((_RB, 128), lambda i: (i, 0)),
            pl.BlockSpec((_RB, 128), lambda i: (i, 0)),
        ],
        out_shape=[
            jax.ShapeDtypeStruct((_EROWS_PAD, 128), jnp.int32),
            jax.ShapeDtypeStruct((_EROWS_PAD, 128), jnp.int32),
        ],
    )(ei4)


def _final_call(hs, agg):
    return pl.pallas_call(
        _final_body,
        grid=(_NBLK,),
        in_specs=[
            pl.BlockSpec((_BN, _HIDDEN), lambda i: (i, 0)),
            pl.BlockSpec((2, _BN, 32), lambda i: (0, i, 0)),
        ],
        out_specs=pl.BlockSpec((_BN, _HIDDEN), lambda i: (i, 0)),
        out_shape=jax.ShapeDtypeStruct((_N, _HIDDEN), jnp.float32),
    )(hs, agg)


def _make_sc_call():
    return pl.kernel(
        _sc_body,
        out_type=jax.ShapeDtypeStruct((2, _AGG_ROWS, 32), jnp.float32),
        mesh=plsc.VectorSubcoreMesh(core_axis_name="c", subcore_axis_name="s"),
        compiler_params=pltpu.CompilerParams(use_tc_tiling_on_sc=False),
        scratch_types=[
            pltpu.VMEM((2, _KB, 128), jnp.int32),
            pltpu.VMEM((_KB, 128), jnp.int32),
            pltpu.VMEM((2, _KB, 128, 32), jnp.float32),
            pltpu.VMEM((128, 32), jnp.float32),
            pltpu.VMEM_SHARED((_AGG_ROWS, 32), jnp.float32),
            pltpu.SemaphoreType.DMA,
        ],
    )


def kernel(x, edge_index, edge_attr, node_type_emb, edge_type_emb,
           W_self, W_msg, W_edge, b):
    f10, ph = _pe_consts()
    b2 = b.reshape(1, _HIDDEN)
    hs, g, mac10, at2 = _node_call(
        x, node_type_emb, edge_type_emb, W_self, W_msg, W_edge, b2, f10, ph)
    g = g.reshape(8 * _N, 32)
    am = jnp.broadcast_to(mac10[:, None, :], (_M, _N, _ATTR_DIM))
    at = at2.reshape(_N, _M, _ATTR_DIM).transpose(1, 0, 2)

    ei = edge_index.reshape(2, _EROWS, 128)
    ea = edge_attr.T.reshape(2, _EROWS, 128)
    ei4 = jnp.pad(jnp.concatenate([ei, ea], axis=0),
                  ((0, 0), (0, _EROWS_PAD - _EROWS), (0, 0)))
    eidx2d, dst2d = _edge_call(ei4)
    eidx2d = jnp.pad(eidx2d, ((0, 8), (0, 0)))

    zz = jnp.zeros((128, 32), jnp.float32)
    agg = _make_sc_call()(g, eidx2d, dst2d, zz)

    out = _final_call(hs, agg)
    return out, (am, at)


# speedup vs baseline: 14.3666x; 1.2378x over previous
import functools
import math

import jax
import jax.numpy as jnp
from jax import lax
from jax.experimental import pallas as pl
from jax.experimental.pallas import tpu as pltpu
from jax.experimental.pallas import tpu_sc as plsc

_ATTR_DIM = 10
_ATTRN = 5
_HIDDEN = 50
_N = 50000
_E = 1600000
_M = 10

_EROWS = _E // 128
_EROWS_PAD = 12544
_TILE_ROWS = _EROWS_PAD // 16
_KB = 2
_SROWS = 8
_SCHUNKS = _SROWS // _KB
_NSUPER = _TILE_ROWS // _SROWS
_AGG_ROWS = 51200
_ZROWS = _AGG_ROWS // 16

_BN = 2000
_NBLK = _N // _BN
_RB = 112
_EBLK = _EROWS_PAD // _RB


def _pe_consts():
    f = [float(i) * (-math.log(10000.0) / _ATTR_DIM) for i in range(0, _ATTR_DIM, 2)]
    f10 = [f[d // 2] for d in range(_ATTR_DIM)]
    ph = [0.0 if d % 2 == 0 else math.pi / 2.0 for d in range(_ATTR_DIM)]
    return (jnp.asarray(f10, jnp.float32).reshape(1, _ATTR_DIM),
            jnp.asarray(ph, jnp.float32).reshape(1, _ATTR_DIM))


def _pack_consts():
    import numpy as np
    f = [float(i) * (-math.log(10000.0) / _ATTR_DIM) for i in range(0, _ATTR_DIM, 2)]
    f10 = [f[d // 2] for d in range(_ATTR_DIM)]
    ph = [0.0 if d % 2 == 0 else math.pi / 2.0 for d in range(_ATTR_DIM)]
    sel = np.zeros((15, 120), np.float32)
    ph120 = np.zeros((1, 120), np.float32)
    for i in range(15):
        for j in range(8):
            sel[i, 8 * i + j] = f10[2 + j]
            ph120[0, 8 * i + j] = ph[2 + j]
    p50 = np.zeros((120, 50), np.float32)
    base50 = np.zeros((1, 50), np.float32)
    for i in range(5):
        base50[0, 10 * i + 1] = 1.0
        for j in range(8):
            p50[8 * i + j, 10 * i + 2 + j] = 1.0
    pt100 = np.zeros((120, 100), np.float32)
    base100 = np.zeros((1, 100), np.float32)
    for i in range(10):
        base100[0, 10 * i + 1] = 1.0
        for j in range(8):
            pt100[40 + 8 * i + j, 10 * i + 2 + j] = 1.0
    return (jnp.asarray(sel), jnp.asarray(ph120), jnp.asarray(p50),
            jnp.asarray(base50), jnp.asarray(pt100), jnp.asarray(base100))


def _node_body(x_ref, emb_ref, eemb_ref, ws_ref, wm_ref, we_ref, b_ref,
               f10_ref, ph_ref, sel_ref, ph120_ref, p50_ref, base50_ref,
               pt100_ref, base100_ref,
               hs_ref, g_ref, mac_ref, at2_ref):
    f10 = f10_ref[...]
    ph = ph_ref[...]
    xb = x_ref[...]
    bn = xb.shape[0]
    ang = jnp.dot(xb, sel_ref[...], preferred_element_type=jnp.float32, precision=lax.Precision.HIGHEST) + ph120_ref[...]
    s = jnp.sin(ang)
    embcat = jnp.concatenate([emb_ref[i:i + 1] for i in range(_ATTRN)], axis=1)
    h = (jnp.dot(s, p50_ref[...], preferred_element_type=jnp.float32, precision=lax.Precision.HIGHEST)
         + base50_ref[...] + embcat)
    hs_ref[...] = jnp.dot(h, ws_ref[...], preferred_element_type=jnp.float32, precision=lax.Precision.HIGHEST)
    hm = jnp.dot(h, wm_ref[...], preferred_element_type=jnp.float32, precision=lax.Precision.HIGHEST)
    pe0 = jnp.sin(ph)
    pe1 = jnp.sin(f10 + ph)
    ea4 = jnp.concatenate([eemb_ref[0:1] + pe0, eemb_ref[0:1] + pe1,
                           eemb_ref[1:2] + pe0, eemb_ref[1:2] + pe1], axis=0)
    c4 = jnp.dot(ea4, we_ref[...], preferred_element_type=jnp.float32, precision=lax.Precision.HIGHEST) + b_ref[...]
    for k in range(4):
        g = jnp.maximum(hm + c4[k:k + 1], 0.0)
        g_ref[0, k] = g[:, 0:32]
        g_ref[1, k] = g[:, 18:50]
    emb4t = jnp.concatenate([emb_ref[4:5]] * _M, axis=1)
    at2_ref[...] = (jnp.dot(s, pt100_ref[...], preferred_element_type=jnp.float32, precision=lax.Precision.HIGHEST)
                    + base100_ref[...] + emb4t)
    mac_ref[...] = jnp.concatenate(
        [jnp.sin(float(i) * f10 + ph) + emb_ref[2] for i in range(_M)], axis=0)


def _edge_body(ei_ref, eidx_ref, dst_ref):
    pid = pl.program_id(0)
    src = ei_ref[0]
    dstv = ei_ref[1]
    a0 = ei_ref[2]
    a1 = ei_ref[3]
    rid = pid * _RB + lax.broadcasted_iota(jnp.int32, (_RB, 128), 0)
    valid = rid < _EROWS
    base = jnp.where(valid, (a0 * 2 + a1) * _N + src, 0)
    eidx_ref[0] = base
    eidx_ref[1] = base + 4 * _N
    dst_ref[...] = jnp.where(valid, dstv, _N)


def _final_body(hs_ref, agg_ref, out_ref):
    cat = jnp.concatenate([agg_ref[0, :, 0:25], agg_ref[1, :, 7:32]], axis=1)
    out_ref[...] = jnp.maximum(hs_ref[...] + cat, 0.0)


def _sc_body(g_hbm, eidx_hbm, dst_hbm, zz_hbm, out_hbm,
             ebig, dbig, rows_v, zz_v, aggm, semg, semi):
    cid = lax.axis_index("c")
    sid = lax.axis_index("s")
    zrow = pl.multiple_of(sid * _ZROWS, 8)
    pltpu.sync_copy(zz_hbm, zz_v)

    def zloop(i, c):
        r = pl.multiple_of(zrow + i * 64, 8)
        pltpu.sync_copy(zz_v, aggm.at[pl.ds(r, 64)])
        return c

    lax.fori_loop(0, _ZROWS // 64, zloop, 0)
    plsc.subcore_barrier()
    row0 = sid * _TILE_ROWS

    def idx_load_sync(buf, srow):
        pltpu.sync_copy(eidx_hbm.at[cid, pl.ds(srow, _SROWS)], ebig.at[buf])
        pltpu.sync_copy(dst_hbm.at[pl.ds(srow, _SROWS)], dbig.at[buf])

    def idx_load_start(buf, srow):
        pltpu.async_copy(eidx_hbm.at[cid, pl.ds(srow, _SROWS)],
                         ebig.at[buf], semi)
        pltpu.async_copy(dst_hbm.at[pl.ds(srow, _SROWS)], dbig.at[buf], semi)

    def idx_load_wait(buf):
        pltpu.make_async_copy(eidx_hbm.at[cid, pl.ds(row0, _SROWS)],
                              ebig.at[buf], semi).wait()
        pltpu.make_async_copy(dst_hbm.at[pl.ds(row0, _SROWS)],
                              dbig.at[buf], semi).wait()

    def fire(rbuf, sbuf, c):
        for j in range(_KB):
            pltpu.async_copy(g_hbm.at[ebig.at[sbuf].at[c * _KB + j]],
                             rows_v.at[rbuf].at[j], semg)

    def drain():
        for j in range(_KB):
            pltpu.make_async_copy(g_hbm.at[pl.ds(0, 128)],
                                  rows_v.at[0].at[j], semg).wait()

    def scatter(rbuf, sbuf, c):
        for j in range(_KB):
            pltpu.sync_copy(rows_v.at[rbuf].at[j],
                            aggm.at[dbig.at[sbuf].at[c * _KB + j]], add=True)

    idx_load_sync(0, row0)
    fire(0, 0, 0)

    def super_body(s, carry):
        sbuf = s & 1
        nbuf = 1 - sbuf
        idx_load_start(nbuf, row0 + (s + 1) * _SROWS)
        for c in range(_SCHUNKS):
            if c + 1 < _SCHUNKS:
                fire((c + 1) & 1, sbuf, c + 1)
            else:
                idx_load_wait(nbuf)
                fire(0, nbuf, 0)
            drain()
            scatter(c & 1, sbuf, c)
        return carry

    lax.fori_loop(0, _NSUPER, super_body, 0)
    drain()
    plsc.subcore_barrier()
    def wloop(i, c):
        r = pl.multiple_of(zrow + i * 64, 8)
        pltpu.sync_copy(aggm.at[pl.ds(r, 64)], zz_v)
        pltpu.sync_copy(zz_v, out_hbm.at[cid, pl.ds(r, 64)])
        return c

    lax.fori_loop(0, _ZROWS // 64, wloop, 0)


def _node_call(x, node_type_emb, edge_type_emb, W_self, W_msg, W_edge, b2, f10, ph):
    sel, ph120, p50, base50, pt100, base100 = _pack_consts()
    full = lambda *s: pl.BlockSpec(s, lambda i: tuple(0 for _ in s))
    return pl.pallas_call(
        _node_body,
        grid=(_NBLK,),
        in_specs=[
            pl.BlockSpec((_BN, 15), lambda i: (i, 0)),
            full(_ATTRN, _ATTR_DIM),
            full(2, _ATTR_DIM),
            full(_HIDDEN, _HIDDEN),
            full(_HIDDEN, _HIDDEN),
            full(_ATTR_DIM, _HIDDEN),
            full(1, _HIDDEN),
            full(1, _ATTR_DIM),
            full(1, _ATTR_DIM),
            full(15, 120),
            full(1, 120),
            full(120, _HIDDEN),
            full(1, _HIDDEN),
            full(120, 100),
            full(1, 100),
        ],
        out_specs=[
            pl.BlockSpec((_BN, _HIDDEN), lambda i: (i, 0)),
            pl.BlockSpec((2, 4, _BN, 32), lambda i: (0, 0, i, 0)),
            pl.BlockSpec((_M, _ATTR_DIM), lambda i: (0, 0)),
            pl.BlockSpec((_BN, 100), lambda i: (i, 0)),
        ],
        out_shape=[
            jax.ShapeDtypeStruct((_N, _HIDDEN), jnp.float32),
            jax.ShapeDtypeStruct((2, 4, _N, 32), jnp.float32),
            jax.ShapeDtypeStruct((_M, _ATTR_DIM), jnp.float32),
            jax.ShapeDtypeStruct((_N, 100), jnp.float32),
        ],
    )(x, node_type_emb, edge_type_emb, W_self, W_msg, W_edge, b2, f10, ph,
      sel, ph120, p50, base50, pt100, base100)


def _edge_call(ei4):
    return pl.pallas_call(
        _edge_body,
        grid=(_EBLK,),
        in_specs=[pl.BlockSpec((4, _RB, 128), lambda i: (0, i, 0))],
        out_specs=[
            pl.BlockSpec((2, _RB, 128), lambda i: (0, i, 0)),
            pl.BlockSpec((_RB, 128), lambda i: (i, 0)),
        ],
        out_shape=[
            jax.ShapeDtypeStruct((2, _EROWS_PAD, 128), jnp.int32),
            jax.ShapeDtypeStruct((_EROWS_PAD, 128), jnp.int32),
        ],
    )(ei4)


def _final_call(hs, agg):
    return pl.pallas_call(
        _final_body,
        grid=(_NBLK,),
        in_specs=[
            pl.BlockSpec((_BN, _HIDDEN), lambda i: (i, 0)),
            pl.BlockSpec((2, _BN, 32), lambda i: (0, i, 0)),
        ],
        out_specs=pl.BlockSpec((_BN, _HIDDEN), lambda i: (i, 0)),
        out_shape=jax.ShapeDtypeStruct((_N, _HIDDEN), jnp.float32),
    )(hs, agg)


def _make_sc_call():
    return pl.kernel(
        _sc_body,
        out_type=jax.ShapeDtypeStruct((2, _AGG_ROWS, 32), jnp.float32),
        mesh=plsc.VectorSubcoreMesh(core_axis_name="c", subcore_axis_name="s"),
        compiler_params=pltpu.CompilerParams(use_tc_tiling_on_sc=False),
        scratch_types=[
            pltpu.VMEM((2, _SROWS, 128), jnp.int32),
            pltpu.VMEM((2, _SROWS, 128), jnp.int32),
            pltpu.VMEM((2, _KB, 128, 32), jnp.float32),
            pltpu.VMEM((64, 32), jnp.float32),
            pltpu.VMEM_SHARED((_AGG_ROWS, 32), jnp.float32),
            pltpu.SemaphoreType.DMA,
            pltpu.SemaphoreType.DMA,
        ],
    )


def kernel(x, edge_index, edge_attr, node_type_emb, edge_type_emb,
           W_self, W_msg, W_edge, b):
    f10, ph = _pe_consts()
    b2 = b.reshape(1, _HIDDEN)
    hs, g, mac10, at2 = _node_call(
        x, node_type_emb, edge_type_emb, W_self, W_msg, W_edge, b2, f10, ph)
    g = g.reshape(8 * _N, 32)
    am = jnp.broadcast_to(mac10[:, None, :], (_M, _N, _ATTR_DIM))
    at = at2.reshape(_N, _M, _ATTR_DIM).transpose(1, 0, 2)

    ei = edge_index.reshape(2, _EROWS, 128)
    ea = edge_attr.T.reshape(2, _EROWS, 128)
    ei4 = jnp.pad(jnp.concatenate([ei, ea], axis=0),
                  ((0, 0), (0, _EROWS_PAD - _EROWS), (0, 0)))
    eidx2d, dst2d = _edge_call(ei4)
    eidx2d = jnp.pad(eidx2d, ((0, 0), (0, _SROWS), (0, 0)))
    dst2d = jnp.pad(dst2d, ((0, _SROWS), (0, 0)))

    zz = jnp.zeros((64, 32), jnp.float32)
    agg = _make_sc_call()(g, eidx2d, dst2d, zz)

    out = _final_call(hs, agg)
    return out, (am, at)
